# speedup vs baseline: 2.5223x; 2.5223x over previous
import jax
import jax.numpy as jnp
from jax import lax
from jax.experimental import pallas as pl
from jax.experimental.pallas import tpu as pltpu
from jax.experimental.pallas import tpu_sc as plsc

N = 100000
D = 128
B = 80
NBLK = N // B
NC = 2
NS = 16
NW = NC * NS
ROUNDS = (NBLK + NW - 1) // NW
LANES = 16
CHUNKS = D // LANES


def _body(x_hbm, din_hbm, dout_hbm, tin_hbm, tout_hbm, out_hbm,
          xbuf, abuf, bbuf, iibuf, iobuf, sem, gsem):
    w = lax.axis_index("s") * NC + lax.axis_index("c")

    def round_body(r, carry):
        bid = r * NW + w

        @pl.when(bid < NBLK)
        def _():
            base = bid * B
            cp_ii = pltpu.async_copy(din_hbm.at[pl.ds(base, B)], iibuf, sem)
            cp_io = pltpu.async_copy(dout_hbm.at[pl.ds(base, B)], iobuf, sem)
            cp_x = pltpu.async_copy(x_hbm.at[pl.ds(base, B)], xbuf, sem)
            cp_ii.wait()
            cp_io.wait()
            g_in = pltpu.async_copy(tin_hbm.at[iibuf], abuf, gsem)
            g_out = pltpu.async_copy(tout_hbm.at[iobuf], bbuf, gsem)
            cp_x.wait()
            g_in.wait()
            g_out.wait()

            def row_body(i, c):
                for cc in range(CHUNKS):
                    s = pl.ds(cc * LANES, LANES)
                    xbuf[i, s] = xbuf[i, s] + abuf[i, s] + bbuf[i, s]
                return c

            lax.fori_loop(0, B, row_body, 0)
            pltpu.sync_copy(xbuf, out_hbm.at[pl.ds(base, B)])

        return carry

    lax.fori_loop(0, ROUNDS, round_body, 0)


@jax.jit
def kernel(x, in_degrees, out_degrees, in_degree_table, out_degree_table):
    mesh = plsc.VectorSubcoreMesh(
        core_axis_name="c", subcore_axis_name="s",
        num_cores=NC, num_subcores=NS,
    )
    f = pl.kernel(
        _body,
        out_type=jax.ShapeDtypeStruct((N, D), jnp.float32),
        mesh=mesh,
        scratch_types=[
            pltpu.VMEM((B, D), jnp.float32),
            pltpu.VMEM((B, D), jnp.float32),
            pltpu.VMEM((B, D), jnp.float32),
            pltpu.VMEM((B,), jnp.int32),
            pltpu.VMEM((B,), jnp.int32),
            pltpu.SemaphoreType.DMA,
            pltpu.SemaphoreType.DMA,
        ],
    )
    return f(x, in_degrees.astype(jnp.int32), out_degrees.astype(jnp.int32),
             in_degree_table, out_degree_table)


# speedup vs baseline: 3.5346x; 1.4014x over previous
import jax
import jax.numpy as jnp
from jax import lax
from jax.experimental import pallas as pl
from jax.experimental.pallas import tpu as pltpu
from jax.experimental.pallas import tpu_sc as plsc

N = 100000
D = 128
B = 80
NBLK = N // B
NC = 2
NS = 16
NW = NC * NS
ROUNDS = (NBLK + NW - 1) // NW
LANES = 16
CHUNKS = D // LANES


def _body(x_hbm, din_hbm, dout_hbm, tin_hbm, tout_hbm, out_hbm,
          xb0, ab0, bb0, ii0, io0, xb1, ab1, bb1, ii1, io1,
          is0, xs0, gs0, os0, is1, xs1, gs1, os1):
    w = lax.axis_index("s") * NC + lax.axis_index("c")
    slot0 = (xb0, ab0, bb0, ii0, io0, is0, xs0, gs0, os0)
    slot1 = (xb1, ab1, bb1, ii1, io1, is1, xs1, gs1, os1)

    def active(r):
        return (r >= 0) & (r < ROUNDS) & (r * NW + w < NBLK)

    def stage(r, s):
        xb, ab, bb, ii, io, isem, xsem, gsem, osem = s
        base = (r * NW + w) * B
        pltpu.async_copy(din_hbm.at[pl.ds(base, B)], ii, isem)
        pltpu.async_copy(dout_hbm.at[pl.ds(base, B)], io, isem)
        pltpu.async_copy(x_hbm.at[pl.ds(base, B)], xb, xsem)

    def wait_idx_issue_gathers(s):
        xb, ab, bb, ii, io, isem, xsem, gsem, osem = s
        pltpu.make_async_copy(din_hbm.at[pl.ds(0, B)], ii, isem).wait()
        pltpu.make_async_copy(dout_hbm.at[pl.ds(0, B)], io, isem).wait()
        pltpu.async_copy(tin_hbm.at[ii], ab, gsem)
        pltpu.async_copy(tout_hbm.at[io], bb, gsem)

    def wait_loads(s):
        xb, ab, bb, ii, io, isem, xsem, gsem, osem = s
        pltpu.make_async_copy(x_hbm.at[pl.ds(0, B)], xb, xsem).wait()
        pltpu.make_async_copy(tin_hbm.at[pl.ds(0, B)], ab, gsem).wait()
        pltpu.make_async_copy(tout_hbm.at[pl.ds(0, B)], bb, gsem).wait()

    def compute_and_scatter(r, s):
        xb, ab, bb, ii, io, isem, xsem, gsem, osem = s

        def row_body(i, c):
            for cc in range(CHUNKS):
                sl = pl.ds(cc * LANES, LANES)
                xb[i, sl] = xb[i, sl] + ab[i, sl] + bb[i, sl]
            return c

        lax.fori_loop(0, B, row_body, 0)
        base = (r * NW + w) * B
        pltpu.async_copy(xb, out_hbm.at[pl.ds(base, B)], osem)

    def wait_scatter(s):
        xb, ab, bb, ii, io, isem, xsem, gsem, osem = s
        pltpu.make_async_copy(xb, out_hbm.at[pl.ds(0, B)], osem).wait()

    def emit_round(r, cur, nxt):
        @pl.when(active(r - 1))
        def _():
            wait_scatter(nxt)

        @pl.when(active(r + 1))
        def _():
            stage(r + 1, nxt)
            wait_idx_issue_gathers(nxt)

        @pl.when(active(r))
        def _():
            wait_loads(cur)
            compute_and_scatter(r, cur)

    @pl.when(active(0))
    def _():
        stage(0, slot0)
        wait_idx_issue_gathers(slot0)

    def pair_body(g, carry):
        emit_round(2 * g, slot0, slot1)
        emit_round(2 * g + 1, slot1, slot0)
        return carry

    lax.fori_loop(0, ROUNDS // 2, pair_body, 0)

    @pl.when(active(ROUNDS - 1))
    def _():
        wait_scatter(slot1)


@jax.jit
def kernel(x, in_degrees, out_degrees, in_degree_table, out_degree_table):
    mesh = plsc.VectorSubcoreMesh(
        core_axis_name="c", subcore_axis_name="s",
        num_cores=NC, num_subcores=NS,
    )
    buf = lambda: pltpu.VMEM((B, D), jnp.float32)
    ibuf = lambda: pltpu.VMEM((B,), jnp.int32)
    f = pl.kernel(
        _body,
        out_type=jax.ShapeDtypeStruct((N, D), jnp.float32),
        mesh=mesh,
        scratch_types=[
            buf(), buf(), buf(), ibuf(), ibuf(),
            buf(), buf(), buf(), ibuf(), ibuf(),
            pltpu.SemaphoreType.DMA, pltpu.SemaphoreType.DMA,
            pltpu.SemaphoreType.DMA, pltpu.SemaphoreType.DMA,
            pltpu.SemaphoreType.DMA, pltpu.SemaphoreType.DMA,
            pltpu.SemaphoreType.DMA, pltpu.SemaphoreType.DMA,
        ],
    )
    return f(x, in_degrees.astype(jnp.int32), out_degrees.astype(jnp.int32),
             in_degree_table, out_degree_table)
